# baseline (speedup 1.0000x reference)
import functools

import jax
import jax.numpy as jnp
from jax import lax
from jax.experimental import pallas as pl
from jax.experimental.pallas import tpu as pltpu
from jax.experimental.pallas import tpu_sc as plsc


def kernel(etypes, table):
    E = etypes.shape[0]
    V, D = table.shape
    assert V * D == 8 * 128

    info = plsc.get_sparse_core_info()
    NC, NS = info.num_cores, info.num_subcores
    NW = NC * NS

    CH = 128
    n_chunks = E // CH
    assert n_chunks * CH == E
    n_main = (n_chunks // NW) & ~1
    n_tail = n_chunks - n_main * NW

    mesh = plsc.VectorSubcoreMesh(core_axis_name="c", subcore_axis_name="s")

    @functools.partial(
        pl.kernel,
        mesh=mesh,
        compiler_params=pltpu.CompilerParams(
            use_tc_tiling_on_sc=True, needs_layout_passes=False
        ),
        out_type=jax.ShapeDtypeStruct((D, E), jnp.float32),
        scratch_types=[
            pltpu.VMEM((8, 128), jnp.float32),
            pltpu.VMEM((CH,), jnp.int32),
            pltpu.VMEM((D, CH), jnp.float32),
            pltpu.VMEM((D, CH), jnp.float32),
            pltpu.SemaphoreType.DMA,
            pltpu.SemaphoreType.DMA,
        ],
    )
    def emb_kernel(
        etypes_hbm, table_hbm, out_hbm, tab_v, eidx_v, trows0, trows1,
        sem0, sem1,
    ):
        wid = lax.axis_index("s") * NC + lax.axis_index("c")
        pltpu.sync_copy(table_hbm, tab_v)

        def fill(c, trows):
            pltpu.sync_copy(etypes_hbm.at[pl.ds(c * CH, CH)], eidx_v)
            rows, col0 = [], []
            for g in range(CH // 16):
                ev = eidx_v[pl.ds(g * 16, 16)]
                rows.append(jnp.right_shift(ev, 1))
                col0.append(jnp.left_shift(jnp.bitwise_and(ev, 1), 6))

            def jbody(j0, carry):
                for jj in range(16):
                    j = j0 * 16 + jj
                    for g in range(CH // 16):
                        vals = plsc.load_gather(tab_v, [rows[g], col0[g] + j])
                        trows[j, pl.ds(g * 16, 16)] = vals
                return carry

            lax.fori_loop(0, D // 16, jbody, 0)

        def dst(c):
            return out_hbm.at[:, pl.ds(c * CH, CH)]

        c0 = wid
        fill(c0, trows0)
        pltpu.async_copy(trows0, dst(c0), sem0)
        c1 = wid + NW
        fill(c1, trows1)
        pltpu.async_copy(trows1, dst(c1), sem1)

        def body(t, carry):
            ca = wid + NW * (2 * t)
            pltpu.make_async_copy(trows0, dst(ca), sem0).wait()
            fill(ca, trows0)
            pltpu.async_copy(trows0, dst(ca), sem0)
            cb = wid + NW * (2 * t + 1)
            pltpu.make_async_copy(trows1, dst(cb), sem1).wait()
            fill(cb, trows1)
            pltpu.async_copy(trows1, dst(cb), sem1)
            return carry

        lax.fori_loop(1, n_main // 2, body, 0)
        pltpu.make_async_copy(trows0, dst(c0), sem0).wait()
        pltpu.make_async_copy(trows1, dst(c1), sem1).wait()

        ct = n_main * NW + wid
        fill(ct, trows0)
        pltpu.async_copy(trows0, dst(ct), sem0).wait()

        @pl.when(wid < n_tail - NW)
        def _extra():
            ce = (n_main + 1) * NW + wid
            fill(ce, trows1)
            pltpu.async_copy(trows1, dst(ce), sem1).wait()

    tab8 = table.reshape(8, 128)
    return emb_kernel(etypes, tab8).T


# speedup vs baseline: 1.2668x; 1.2668x over previous
import functools

import jax
import jax.numpy as jnp
from jax import lax
from jax.experimental import pallas as pl
from jax.experimental.pallas import tpu as pltpu
from jax.experimental.pallas import tpu_sc as plsc


def kernel(etypes, table):
    E = etypes.shape[0]
    V, D = table.shape
    assert V * D == 8 * 128

    info = plsc.get_sparse_core_info()
    NC, NS = info.num_cores, info.num_subcores
    NW = NC * NS

    CH = 128
    n_chunks = E // CH
    assert n_chunks * CH == E
    n_main = (n_chunks // NW) & ~1
    n_tail = n_chunks - n_main * NW

    mesh = plsc.VectorSubcoreMesh(core_axis_name="c", subcore_axis_name="s")

    @functools.partial(
        pl.kernel,
        mesh=mesh,
        compiler_params=pltpu.CompilerParams(
            use_tc_tiling_on_sc=True, needs_layout_passes=False
        ),
        out_type=jax.ShapeDtypeStruct((D, E), jnp.float32),
        scratch_types=[
            pltpu.VMEM((8, 128), jnp.float32),
            pltpu.VMEM((CH,), jnp.int32),
            pltpu.VMEM((D, CH), jnp.float32),
            pltpu.VMEM((D, CH), jnp.float32),
            pltpu.SemaphoreType.DMA,
            pltpu.SemaphoreType.DMA,
        ],
    )
    def emb_kernel(
        etypes_hbm, table_hbm, out_hbm, tab_v, eidx_v, trows0, trows1,
        sem0, sem1,
    ):
        wid = lax.axis_index("s") * NC + lax.axis_index("c")
        pltpu.sync_copy(table_hbm, tab_v)

        def fill(c, trows):
            pltpu.sync_copy(etypes_hbm.at[pl.ds(c * CH, CH)], eidx_v)
            rows, col0 = [], []
            for g in range(CH // 16):
                ev = eidx_v[pl.ds(g * 16, 16)]
                rows.append(jnp.right_shift(ev, 1))
                col0.append(jnp.left_shift(jnp.bitwise_and(ev, 1), 6))

            def jbody(j0, carry):
                vals = []
                for jj in range(4):
                    j = j0 * 4 + jj
                    for g in range(CH // 16):
                        vals.append(
                            plsc.load_gather(tab_v, [rows[g], col0[g] + j])
                        )
                k = 0
                for jj in range(4):
                    j = j0 * 4 + jj
                    for g in range(CH // 16):
                        trows[j, pl.ds(g * 16, 16)] = vals[k]
                        k += 1
                return carry

            lax.fori_loop(0, D // 4, jbody, 0)

        def dst(c):
            return out_hbm.at[:, pl.ds(c * CH, CH)]

        c0 = wid
        fill(c0, trows0)
        pltpu.async_copy(trows0, dst(c0), sem0)
        c1 = wid + NW
        fill(c1, trows1)
        pltpu.async_copy(trows1, dst(c1), sem1)

        def body(t, carry):
            ca = wid + NW * (2 * t)
            pltpu.make_async_copy(trows0, dst(ca), sem0).wait()
            fill(ca, trows0)
            pltpu.async_copy(trows0, dst(ca), sem0)
            cb = wid + NW * (2 * t + 1)
            pltpu.make_async_copy(trows1, dst(cb), sem1).wait()
            fill(cb, trows1)
            pltpu.async_copy(trows1, dst(cb), sem1)
            return carry

        lax.fori_loop(1, n_main // 2, body, 0)
        pltpu.make_async_copy(trows0, dst(c0), sem0).wait()
        pltpu.make_async_copy(trows1, dst(c1), sem1).wait()

        ct = n_main * NW + wid
        fill(ct, trows0)
        pltpu.async_copy(trows0, dst(ct), sem0).wait()

        @pl.when(wid < n_tail - NW)
        def _extra():
            ce = (n_main + 1) * NW + wid
            fill(ce, trows1)
            pltpu.async_copy(trows1, dst(ce), sem1).wait()

    tab8 = table.reshape(8, 128)
    return emb_kernel(etypes, tab8).T


# speedup vs baseline: 6.5540x; 5.1736x over previous
import functools

import jax
import jax.numpy as jnp
from jax import lax
from jax.experimental import pallas as pl
from jax.experimental.pallas import tpu as pltpu
from jax.experimental.pallas import tpu_sc as plsc


def kernel(etypes, table):
    E = etypes.shape[0]
    V, D = table.shape
    assert V * D == 8 * 128

    info = plsc.get_sparse_core_info()
    NC, NS = info.num_cores, info.num_subcores
    NW = NC * NS

    CH = 128
    n_chunks = E // CH
    assert n_chunks * CH == E
    n_main = (n_chunks // NW) & ~1
    n_tail = n_chunks - n_main * NW

    mesh = plsc.VectorSubcoreMesh(core_axis_name="c", subcore_axis_name="s")

    @functools.partial(
        pl.kernel,
        mesh=mesh,
        compiler_params=pltpu.CompilerParams(
            use_tc_tiling_on_sc=True, needs_layout_passes=False
        ),
        out_type=jax.ShapeDtypeStruct((D, E), jnp.float32),
        scratch_types=[
            pltpu.VMEM((V * D,), jnp.float32),
            pltpu.VMEM((CH,), jnp.int32),
            pltpu.VMEM((D, CH), jnp.float32),
            pltpu.VMEM((D, CH), jnp.float32),
            pltpu.SemaphoreType.DMA,
            pltpu.SemaphoreType.DMA,
        ],
    )
    def emb_kernel(
        etypes_hbm, table_hbm, out_hbm, tab_v, eidx_v, trows0, trows1,
        sem0, sem1,
    ):
        wid = lax.axis_index("s") * NC + lax.axis_index("c")
        pltpu.sync_copy(table_hbm, tab_v)

        def fill(c, trows):
            pltpu.sync_copy(etypes_hbm.at[pl.ds(c * CH, CH)], eidx_v)
            evs = [eidx_v[pl.ds(g * 16, 16)] for g in range(CH // 16)]

            def jbody(j0, carry):
                vals = []
                for jj in range(4):
                    j = j0 * 4 + jj
                    tcol = tab_v[pl.ds(j * 16, 16)]
                    for g in range(CH // 16):
                        vals.append(
                            tcol.at[evs[g]].get(mode="promise_in_bounds")
                        )
                k = 0
                for jj in range(4):
                    j = j0 * 4 + jj
                    for g in range(CH // 16):
                        trows[j, pl.ds(g * 16, 16)] = vals[k]
                        k += 1
                return carry

            lax.fori_loop(0, D // 4, jbody, 0)

        def dst(c):
            return out_hbm.at[:, pl.ds(c * CH, CH)]

        c0 = wid
        fill(c0, trows0)
        pltpu.async_copy(trows0, dst(c0), sem0)
        c1 = wid + NW
        fill(c1, trows1)
        pltpu.async_copy(trows1, dst(c1), sem1)

        def body(t, carry):
            ca = wid + NW * (2 * t)
            pltpu.make_async_copy(trows0, dst(ca), sem0).wait()
            fill(ca, trows0)
            pltpu.async_copy(trows0, dst(ca), sem0)
            cb = wid + NW * (2 * t + 1)
            pltpu.make_async_copy(trows1, dst(cb), sem1).wait()
            fill(cb, trows1)
            pltpu.async_copy(trows1, dst(cb), sem1)
            return carry

        lax.fori_loop(1, n_main // 2, body, 0)
        pltpu.make_async_copy(trows0, dst(c0), sem0).wait()
        pltpu.make_async_copy(trows1, dst(c1), sem1).wait()

        ct = n_main * NW + wid
        fill(ct, trows0)
        pltpu.async_copy(trows0, dst(ct), sem0).wait()

        @pl.when(wid < n_tail - NW)
        def _extra():
            ce = (n_main + 1) * NW + wid
            fill(ce, trows1)
            pltpu.async_copy(trows1, dst(ce), sem1).wait()

    tab_t = table.T.reshape(-1)
    return emb_kernel(etypes, tab_t).T


# speedup vs baseline: 12.4125x; 1.8939x over previous
import functools

import jax
import jax.numpy as jnp
from jax import lax
from jax.experimental import pallas as pl
from jax.experimental.pallas import tpu as pltpu
from jax.experimental.pallas import tpu_sc as plsc


def kernel(etypes, table):
    E = etypes.shape[0]
    V, D = table.shape
    assert V * D == 8 * 128

    info = plsc.get_sparse_core_info()
    NC, NS = info.num_cores, info.num_subcores
    NW = NC * NS

    CH = 128
    n_chunks = E // CH
    assert n_chunks * CH == E
    n_main = (n_chunks // NW) & ~1
    n_tail = n_chunks - n_main * NW

    mesh = plsc.VectorSubcoreMesh(core_axis_name="c", subcore_axis_name="s")

    @functools.partial(
        pl.kernel,
        mesh=mesh,
        compiler_params=pltpu.CompilerParams(
            use_tc_tiling_on_sc=True, needs_layout_passes=False
        ),
        out_type=jax.ShapeDtypeStruct((D, E), jnp.float32),
        scratch_types=[
            pltpu.VMEM((V * D,), jnp.float32),
            pltpu.VMEM((CH,), jnp.int32),
            pltpu.VMEM((CH,), jnp.int32),
            pltpu.VMEM((D, CH), jnp.float32),
            pltpu.VMEM((D, CH), jnp.float32),
            pltpu.SemaphoreType.DMA,
            pltpu.SemaphoreType.DMA,
            pltpu.SemaphoreType.DMA,
            pltpu.SemaphoreType.DMA,
        ],
    )
    def emb_kernel(
        etypes_hbm, table_hbm, out_hbm, tab_v, eidx0, eidx1, trows0, trows1,
        sem0, sem1, semi0, semi1,
    ):
        wid = lax.axis_index("s") * NC + lax.axis_index("c")
        pltpu.sync_copy(table_hbm, tab_v)

        def eslice(c):
            return etypes_hbm.at[pl.ds(c * CH, CH)]

        def fill(c, trows, eidx, semi):
            pltpu.make_async_copy(eslice(c), eidx, semi).wait()
            evs = [eidx[pl.ds(g * 16, 16)] for g in range(CH // 16)]
            cp = jnp.where(c + 2 * NW < n_chunks, c + 2 * NW, 0)
            pltpu.async_copy(eslice(cp), eidx, semi)

            def jbody(j0, carry):
                vals = []
                for jj in range(4):
                    j = j0 * 4 + jj
                    tcol = tab_v[pl.ds(j * 16, 16)]
                    for g in range(CH // 16):
                        vals.append(
                            tcol.at[evs[g]].get(mode="promise_in_bounds")
                        )
                k = 0
                for jj in range(4):
                    j = j0 * 4 + jj
                    for g in range(CH // 16):
                        trows[j, pl.ds(g * 16, 16)] = vals[k]
                        k += 1
                return carry

            lax.fori_loop(0, D // 4, jbody, 0)

        def dst(c):
            return out_hbm.at[:, pl.ds(c * CH, CH)]

        c0 = wid
        c1 = wid + NW
        pltpu.async_copy(eslice(c0), eidx0, semi0)
        pltpu.async_copy(eslice(c1), eidx1, semi1)
        fill(c0, trows0, eidx0, semi0)
        pltpu.async_copy(trows0, dst(c0), sem0)
        fill(c1, trows1, eidx1, semi1)
        pltpu.async_copy(trows1, dst(c1), sem1)

        def body(t, carry):
            ca = wid + NW * (2 * t)
            pltpu.make_async_copy(trows0, dst(ca), sem0).wait()
            fill(ca, trows0, eidx0, semi0)
            pltpu.async_copy(trows0, dst(ca), sem0)
            cb = wid + NW * (2 * t + 1)
            pltpu.make_async_copy(trows1, dst(cb), sem1).wait()
            fill(cb, trows1, eidx1, semi1)
            pltpu.async_copy(trows1, dst(cb), sem1)
            return carry

        lax.fori_loop(1, n_main // 2, body, 0)
        pltpu.make_async_copy(trows0, dst(c0), sem0).wait()
        pltpu.make_async_copy(trows1, dst(c1), sem1).wait()

        ct = n_main * NW + wid
        fill(ct, trows0, eidx0, semi0)
        pltpu.async_copy(trows0, dst(ct), sem0).wait()

        @pl.when(wid < n_tail - NW)
        def _extra():
            ce = (n_main + 1) * NW + wid
            fill(ce, trows1, eidx1, semi1)
            pltpu.async_copy(trows1, dst(ce), sem1).wait()

        pltpu.make_async_copy(eslice(c0), eidx0, semi0).wait()
        pltpu.make_async_copy(eslice(c1), eidx1, semi1).wait()

    tab_t = table.T.reshape(-1)
    return emb_kernel(etypes, tab_t).T
